# baseline (speedup 1.0000x reference)
import functools

import jax
import jax.numpy as jnp
from jax import lax
from jax.experimental import pallas as pl
from jax.experimental.pallas import tpu as pltpu
from jax.experimental.pallas import tpu_sc as plsc

_IMG = 512.0
_MAX_OUT = 100
_IOU_T = 0.5
_SCORE_T = 0.5
_MIN_AREA = 100.0
_N = 20000
_NPAD = 20480
_NSUB = 16
_PER = _NPAD // _NSUB
_SL = _PER // 16

_mesh = plsc.VectorSubcoreMesh(
    core_axis_name="c", subcore_axis_name="s", num_cores=1
)


_scratch_types = [
    pltpu.VMEM((_PER,), jnp.float32),
    pltpu.VMEM((_PER,), jnp.float32),
    pltpu.VMEM((_PER,), jnp.float32),
    pltpu.VMEM((_PER,), jnp.float32),
    pltpu.VMEM((_PER,), jnp.float32),
    pltpu.VMEM((_PER,), jnp.float32),
    pltpu.VMEM((16,), jnp.float32),
    pltpu.VMEM((_NSUB * 16,), jnp.float32),
    pltpu.VMEM((_MAX_OUT, 16), jnp.float32),
    pltpu.VMEM_SHARED((_NSUB * 16,), jnp.float32),
]


def _sc_nms_body(y1h, x1h, y2h, x2h, sh, outh,
            by1, bx1, by2, bx2, cbuf, sfill, pub, call_, outbuf, shc):
    wid = lax.axis_index("s")
    base = wid * _PER
    li = lax.broadcasted_iota(jnp.int32, (16,), 0)

    pltpu.sync_copy(y1h.at[pl.ds(base, _PER)], by1)
    pltpu.sync_copy(x1h.at[pl.ds(base, _PER)], bx1)
    pltpu.sync_copy(y2h.at[pl.ds(base, _PER)], by2)
    pltpu.sync_copy(x2h.at[pl.ds(base, _PER)], bx2)
    pltpu.sync_copy(sh.at[pl.ds(base, _PER)], sfill)

    def init_j(j, carry):
        sl = pl.ds(j * 16, 16)
        v1 = by1[sl]
        u1 = bx1[sl]
        v2 = by2[sl]
        u2 = bx2[sl]
        sv = sfill[sl]
        hh = v2 - v1
        ww = u2 - u1
        area = hh * ww
        valid = (
            (ww / _IMG <= 1.0)
            & (hh / _IMG <= 1.0)
            & (area > _MIN_AREA)
            & (sv >= _SCORE_T)
        )
        s0 = jnp.where(valid, sv, -1.0)
        sfill[sl] = s0
        cbuf[sl] = jnp.where(s0 > 0.0, s0, (s0 - 1.0) * 0.25)
        return carry

    lax.fori_loop(0, _SL, init_j, 0)

    def round_t(t, carry):
        def amax_j(j, c):
            bv, bi = c
            v = cbuf[pl.ds(j * 16, 16)]
            idxv = j * 16 + li
            take = v > bv
            return (jnp.where(take, v, bv), jnp.where(take, idxv, bi))

        bv, bi = lax.fori_loop(
            0,
            _SL,
            amax_j,
            (
                jnp.full((16,), -1e30, jnp.float32),
                jnp.zeros((16,), jnp.int32),
            ),
        )
        mloc = jnp.max(bv)
        iloc = jnp.min(jnp.where(bv == mloc, bi, jnp.int32(2**30)))
        idxf = jnp.full((16,), iloc, jnp.int32)
        gy1 = plsc.load_gather(by1, [idxf])
        gx1 = plsc.load_gather(bx1, [idxf])
        gy2 = plsc.load_gather(by2, [idxf])
        gx2 = plsc.load_gather(bx2, [idxf])
        rec = jnp.where(li == 0, mloc, 0.0)
        rec = jnp.where(li == 1, gy1, rec)
        rec = jnp.where(li == 2, gx1, rec)
        rec = jnp.where(li == 3, gy2, rec)
        rec = jnp.where(li == 4, gx2, rec)
        pub[...] = rec
        pltpu.sync_copy(pub, shc.at[pl.ds(wid * 16, 16)])
        plsc.subcore_barrier()
        pltpu.sync_copy(shc, call_)
        plsc.subcore_barrier()
        vals = plsc.load_gather(call_, [li * 16])
        m = jnp.max(vals)
        wwin = jnp.min(jnp.where(vals == m, li, jnp.int32(2**30)))
        wv = jnp.full((16,), wwin, jnp.int32)
        wy1 = plsc.load_gather(call_, [wv * 16 + 1])
        wx1 = plsc.load_gather(call_, [wv * 16 + 2])
        wy2 = plsc.load_gather(call_, [wv * 16 + 3])
        wx2 = plsc.load_gather(call_, [wv * 16 + 4])

        def sup_j(j, c):
            sl = pl.ds(j * 16, 16)
            v1 = by1[sl]
            u1 = bx1[sl]
            v2 = by2[sl]
            u2 = bx2[sl]
            iy1 = jnp.maximum(wy1, v1)
            ix1 = jnp.maximum(wx1, u1)
            iy2 = jnp.minimum(wy2, v2)
            ix2 = jnp.minimum(wx2, u2)
            inter = jnp.maximum(iy2 - iy1, 0.0) * jnp.maximum(
                ix2 - ix1, 0.0
            )
            a1 = (wy2 - wy1) * (wx2 - wx1)
            a2 = (v2 - v1) * (u2 - u1)
            union = a1 + a2 - inter
            iou = inter / jnp.maximum(union, 1e-8)
            cv = cbuf[sl]
            sfv = sfill[sl]
            cbuf[sl] = jnp.where(
                (iou > _IOU_T) & (cv > 0.0), (sfv - 1.0) * 0.25, cv
            )
            return c

        lax.fori_loop(0, _SL, sup_j, 0)

        @pl.when(wid == wwin)
        def _():
            plsc.store_scatter(
                cbuf,
                [idxf],
                jnp.full((16,), -1.0, jnp.float32),
                mask=(li == 0),
            )
            plsc.store_scatter(
                sfill,
                [idxf],
                jnp.full((16,), -3.0, jnp.float32),
                mask=(li == 0),
            )

        @pl.when(wid == 0)
        def _():
            sc = jnp.where(m > 0.0, m, 0.0)
            row = jnp.where(li == 0, jnp.clip(wy1, 0.0, _IMG), 0.0)
            row = jnp.where(li == 1, jnp.clip(wx1, 0.0, _IMG), row)
            row = jnp.where(li == 2, jnp.clip(wy2, 0.0, _IMG), row)
            row = jnp.where(li == 3, jnp.clip(wx2, 0.0, _IMG), row)
            row = jnp.where(li == 4, sc, row)
            plsc.store_scatter(
                outbuf, [jnp.full((16,), t, jnp.int32), li], row
            )

        return carry

    lax.fori_loop(0, _MAX_OUT, round_t, 0)

    @pl.when(wid == 0)
    def _():
        pltpu.sync_copy(outbuf, outh)


_sc_nms = pl.kernel(
    _sc_nms_body,
    mesh=_mesh,
    out_type=jax.ShapeDtypeStruct((_MAX_OUT, 16), jnp.float32),
    scratch_types=_scratch_types,
    compiler_params=pltpu.CompilerParams(needs_layout_passes=False),
)


@jax.jit
def kernel(boxes, scores):
    pad = _NPAD - _N
    b = jnp.pad(boxes, ((0, pad), (0, 0)))
    s = jnp.pad(scores, ((0, pad),))
    out = _sc_nms(b[:, 0], b[:, 1], b[:, 2], b[:, 3], s)
    return out[:, :5]
